# Initial kernel scaffold; baseline (speedup 1.0000x reference)
import jax
import jax.numpy as jnp
from jax.experimental import pallas as pl


def kernel(f_p, f_ms):
    raise NotImplementedError("write your pallas kernel here")


# speedup vs baseline: 14.2457x; 14.2457x over previous
import functools

import jax
import jax.numpy as jnp
from jax import lax
from jax.experimental import pallas as pl
from jax.experimental.pallas import tpu as pltpu
from jax.experimental.pallas import tpu_sc as plsc

_B, _C, _H, _W = 4, 96, 224, 224
_HW = _H * _W
_N = 256
_UW = 272


def _quant(x, mn, mx):
    return ((x - mn) / (mx - mn) * 255.0).astype(jnp.int32)


def _g(v):
    x = v * jnp.float32(1.0 / 65536.0)
    return x * jnp.log(x + jnp.float32(1e-8))


def _g_tilde(v):
    x = v * jnp.float32(1.0 / 65536.0)
    return x * jnp.log(jnp.abs(x + jnp.float32(1e-8)))


_HCH = 16
_NST = _H // _HCH


def _k1_body(fp_ref, fms_ref, xpa_ref, xms_ref, mw_ref, acc_ref):
    k = pl.program_id(0)
    acc_ref[:, pl.ds(k * _HCH, _HCH), :] = (
        jnp.sum(fp_ref[...], axis=1) / jnp.float32(_C))

    @pl.when(k == _NST - 1)
    def _():
        r16 = lax.broadcasted_iota(jnp.int32, (16, _H), 0)
        c16 = lax.broadcasted_iota(jnp.int32, (16, _H), 1)
        P = (lax.div(c16, 14) == r16).astype(jnp.float32)
        PT = (lax.div(lax.broadcasted_iota(jnp.int32, (_H, 16), 0), 14)
              == lax.broadcasted_iota(jnp.int32, (_H, 16), 1)).astype(
                  jnp.float32)
        for b in range(_B):
            fm = acc_ref[b]
            t = lax.dot_general(P, fm, (((1,), (0,)), ((), ())),
                                preferred_element_type=jnp.float32)
            fa = lax.dot_general(t, PT, (((1,), (0,)), ((), ())),
                                 preferred_element_type=jnp.float32)
            fa = fa * jnp.float32(1.0 / 196.0)
            xpa_ref[b] = _quant(fa, jnp.min(fa), jnp.max(fa))
        fms = fms_ref[...]
        mn = jnp.min(fms, axis=1, keepdims=True)
        mx = jnp.max(fms, axis=1, keepdims=True)
        xms_ref[...] = _quant(fms, mn, mx)
        pa = xpa_ref[...]
        bins4 = lax.broadcasted_iota(jnp.int32, (1, 1, 1, _N), 3)
        eq4 = (pa[:, :, :, None] == bins4).astype(jnp.int32)
        cp = jnp.sum(jnp.sum(eq4, axis=2), axis=1)
        sbins = lax.broadcasted_iota(jnp.int32, (1, 1, _UW), 2)
        w = jnp.sum((cp[:, :, None] == sbins).astype(jnp.int32),
                    axis=1).astype(jnp.float32)
        s1 = lax.broadcasted_iota(jnp.int32, (_UW, _UW), 0).astype(jnp.float32)
        s2 = lax.broadcasted_iota(jnp.int32, (_UW, _UW), 1).astype(jnp.float32)
        M = _g_tilde(jnp.float32(_N) - s1 - s2)
        mw_ref[...] = lax.dot_general(w, M, (((1,), (0,)), ((), ())),
                                      preferred_element_type=jnp.float32)


def _k1(f_p, f_ms_r):
    return pl.pallas_call(
        _k1_body,
        grid=(_NST,),
        in_specs=[pl.BlockSpec((_B, _C, _HCH, _W), lambda k: (0, 0, k, 0)),
                  pl.BlockSpec(memory_space=pltpu.VMEM)],
        out_specs=[pl.BlockSpec(memory_space=pltpu.VMEM),
                   pl.BlockSpec(memory_space=pltpu.VMEM),
                   pl.BlockSpec(memory_space=pltpu.VMEM)],
        out_shape=[jax.ShapeDtypeStruct((_B, 16, 16), jnp.int32),
                   jax.ShapeDtypeStruct((_B * _C, _N), jnp.int32),
                   jax.ShapeDtypeStruct((_B, _UW), jnp.float32)],
        scratch_shapes=[pltpu.VMEM((_B, _H, _W), jnp.float32)],
    )(f_p, f_ms_r)


_NWK = 32
_RPW = (_B * _C) // _NWK
_JSZ = 65536


def _sc_body(xms_hbm, xpa_hbm, mw_hbm, a_hbm, c_hbm, main_hbm,
             xms_v, xpa_v, idxA, idxB, av, cv, cmv, gv, mwv,
             onesv, negv, zerosv, joint_s, cm_s, mw_s):
    cid = lax.axis_index("c")
    sid = lax.axis_index("s")
    wid = sid * 2 + cid

    one16 = jnp.full((16,), 1, jnp.int32)
    zero16 = jnp.full((16,), 0, jnp.int32)
    for k in range(8):
        onesv[pl.ds(k * 16, 16)] = one16
        negv[pl.ds(k * 16, 16)] = -one16

    def zbody(i, carry):
        zerosv[pl.ds(i * 16, 16)] = zero16
        return carry
    lax.fori_loop(0, 256, zbody, 0)

    base_j = sid * _JSZ
    base_cm = sid * 256

    def jz(i, carry):
        pltpu.sync_copy(zerosv, joint_s.at[pl.ds(base_j + i * 4096, 4096)])
        return carry
    lax.fori_loop(0, _JSZ // 4096, jz, 0)
    pltpu.sync_copy(zerosv.at[pl.ds(0, 256)], cm_s.at[pl.ds(base_cm, 256)])

    pltpu.sync_copy(xpa_hbm.at[pl.ds((wid // 8) * _N, _N)], xpa_v)
    mw_base = sid * _UW
    pltpu.sync_copy(mw_hbm.at[pl.ds((wid // 8) * _UW, _UW)], mwv)
    pltpu.sync_copy(mwv, mw_s.at[pl.ds(mw_base, _UW)])

    def row_body(r, carry):
        row = wid * _RPW + r
        pltpu.sync_copy(xms_hbm.at[pl.ds(row * _N, _N)], xms_v)
        for k in range(8):
            idxA[pl.ds(k * 16, 16)] = xms_v[pl.ds(k * 16, 16)] + base_cm
            idxB[pl.ds(k * 16, 16)] = xms_v[pl.ds(128 + k * 16, 16)] + base_cm
        pltpu.sync_copy(onesv, cm_s.at[idxA], add=True)
        pltpu.sync_copy(onesv, cm_s.at[idxB], add=True)
        pltpu.sync_copy(cm_s.at[idxA], av.at[pl.ds(0, 128)])
        pltpu.sync_copy(cm_s.at[idxB], av.at[pl.ds(128, 128)])
        pltpu.sync_copy(av, a_hbm.at[pl.ds(row * _N, _N)])
        pltpu.sync_copy(cm_s.at[pl.ds(base_cm, 256)], cmv)
        for k in range(8):
            idxA[pl.ds(k * 16, 16)] = cmv[pl.ds(k * 16, 16)] + mw_base
            idxB[pl.ds(k * 16, 16)] = cmv[pl.ds(128 + k * 16, 16)] + mw_base
        pltpu.sync_copy(mw_s.at[idxA], gv.at[pl.ds(0, 128)])
        pltpu.sync_copy(mw_s.at[idxB], gv.at[pl.ds(128, 128)])
        accm = gv[pl.ds(0, 16)]
        for k in range(1, 16):
            accm = accm + gv[pl.ds(k * 16, 16)]
        mwv[pl.ds(16, 16)] = accm
        pltpu.sync_copy(mwv.at[pl.ds(16, 16)],
                        main_hbm.at[pl.ds(row * 16, 16)])
        pltpu.sync_copy(zerosv.at[pl.ds(0, 256)], cm_s.at[pl.ds(base_cm, 256)])
        for k in range(8):
            idxA[pl.ds(k * 16, 16)] = (xms_v[pl.ds(k * 16, 16)] * 256
                                       + xpa_v[pl.ds(k * 16, 16)] + base_j)
            idxB[pl.ds(k * 16, 16)] = (xms_v[pl.ds(128 + k * 16, 16)] * 256
                                       + xpa_v[pl.ds(128 + k * 16, 16)] + base_j)
        pltpu.sync_copy(onesv, joint_s.at[idxA], add=True)
        pltpu.sync_copy(onesv, joint_s.at[idxB], add=True)
        pltpu.sync_copy(joint_s.at[idxA], cv.at[pl.ds(0, 128)])
        pltpu.sync_copy(joint_s.at[idxB], cv.at[pl.ds(128, 128)])
        pltpu.sync_copy(cv, c_hbm.at[pl.ds(row * _N, _N)])
        pltpu.sync_copy(negv, joint_s.at[idxA], add=True)
        pltpu.sync_copy(negv, joint_s.at[idxB], add=True)
        return carry
    lax.fori_loop(0, _RPW, row_body, 0)


def _sc_counts(x_ms, x_pa, mw):
    mesh = plsc.VectorSubcoreMesh(core_axis_name="c", subcore_axis_name="s")
    f = pl.kernel(
        _sc_body,
        out_type=[jax.ShapeDtypeStruct((_B * _C * _N,), jnp.int32),
                  jax.ShapeDtypeStruct((_B * _C * _N,), jnp.int32),
                  jax.ShapeDtypeStruct((_B * _C * 16,), jnp.float32)],
        mesh=mesh,
        scratch_types=[
            pltpu.VMEM((_N,), jnp.int32),
            pltpu.VMEM((_N,), jnp.int32),
            pltpu.VMEM((128,), jnp.int32),
            pltpu.VMEM((128,), jnp.int32),
            pltpu.VMEM((_N,), jnp.int32),
            pltpu.VMEM((_N,), jnp.int32),
            pltpu.VMEM((_N,), jnp.int32),
            pltpu.VMEM((_N,), jnp.float32),
            pltpu.VMEM((_UW,), jnp.float32),
            pltpu.VMEM((128,), jnp.int32),
            pltpu.VMEM((128,), jnp.int32),
            pltpu.VMEM((4096,), jnp.int32),
            pltpu.VMEM_SHARED((16 * _JSZ,), jnp.int32),
            pltpu.VMEM_SHARED((16 * 256,), jnp.int32),
            pltpu.VMEM_SHARED((16 * _UW,), jnp.float32),
        ],
    )
    return f(x_ms, x_pa, mw)


def _final_body(a_ref, c_ref, mp_ref, xpa_ref, fms_ref, o_ref):
    af = a_ref[...].astype(jnp.float32)
    h_ms = -(1.0 / _N) * jnp.sum(jnp.log(af * (1.0 / _N) + 1e-8), axis=2)

    pa = xpa_ref[...]
    bq = jnp.sum((pa[:, :, None] == pa[:, None, :]).astype(jnp.int32), axis=2)

    main = jnp.sum(mp_ref[...], axis=2)

    cf = c_ref[...].astype(jnp.float32)
    sf = af + bq[:, None, :].astype(jnp.float32)
    histic = jnp.float32(_N) - sf + 2.0 * cf
    corr = jnp.sum((_g(histic) - _g_tilde(jnp.float32(_N) - sf)) / cf, axis=2)
    h_ms_p = -(main + corr)

    mi = h_ms - h_ms_p
    mx = jnp.max(mi, axis=1, keepdims=True)
    e = jnp.exp(mi - mx)
    soft = e / jnp.sum(e, axis=1, keepdims=True)
    o_ref[...] = fms_ref[...] * (1.0 + soft[:, :, None])


def _final(a3, c3, mp3, x_pa, f_ms3):
    return pl.pallas_call(
        _final_body,
        in_specs=[pl.BlockSpec(memory_space=pltpu.VMEM)] * 5,
        out_specs=pl.BlockSpec(memory_space=pltpu.VMEM),
        out_shape=jax.ShapeDtypeStruct((_B, _C, _N), jnp.float32),
    )(a3, c3, mp3, x_pa, f_ms3)


def kernel(f_p, f_ms):
    B, C, H, W = f_p.shape
    x_pa, x_ms, mw = _k1(f_p, f_ms.reshape(B * C, _N))
    a, c, mainp = _sc_counts(x_ms.reshape(B * C * _N),
                             x_pa.reshape(B * _N), mw.reshape(B * _UW))
    rel = _final(a.reshape(B, C, _N), c.reshape(B, C, _N),
                 mainp.reshape(B, C, 16), x_pa.reshape(B, _N),
                 f_ms.reshape(B, C, _N))
    return rel.reshape(B, C, 16, 16)


# speedup vs baseline: 16.9452x; 1.1895x over previous
import functools

import jax
import jax.numpy as jnp
from jax import lax
from jax.experimental import pallas as pl
from jax.experimental.pallas import tpu as pltpu
from jax.experimental.pallas import tpu_sc as plsc

_B, _C, _H, _W = 4, 96, 224, 224
_HW = _H * _W
_N = 256
_UW = 272


def _quant(x, mn, mx):
    return ((x - mn) / (mx - mn) * 255.0).astype(jnp.int32)


def _g(v):
    x = v * jnp.float32(1.0 / 65536.0)
    return x * jnp.log(x + jnp.float32(1e-8))


def _g_tilde(v):
    x = v * jnp.float32(1.0 / 65536.0)
    return x * jnp.log(jnp.abs(x + jnp.float32(1e-8)))


_HCH = 16
_NST = _H // _HCH


def _k1_body(fp_ref, fms_ref, xpa_ref, xms_ref, mw_ref, acc_ref):
    k = pl.program_id(0)
    acc_ref[:, pl.ds(k * _HCH, _HCH), :] = (
        jnp.sum(fp_ref[...], axis=1) / jnp.float32(_C))

    @pl.when(k == _NST - 1)
    def _():
        r16 = lax.broadcasted_iota(jnp.int32, (16, _H), 0)
        c16 = lax.broadcasted_iota(jnp.int32, (16, _H), 1)
        P = (lax.div(c16, 14) == r16).astype(jnp.float32)
        PT = (lax.div(lax.broadcasted_iota(jnp.int32, (_H, 16), 0), 14)
              == lax.broadcasted_iota(jnp.int32, (_H, 16), 1)).astype(
                  jnp.float32)
        for b in range(_B):
            fm = acc_ref[b]
            t = lax.dot_general(P, fm, (((1,), (0,)), ((), ())),
                                preferred_element_type=jnp.float32)
            fa = lax.dot_general(t, PT, (((1,), (0,)), ((), ())),
                                 preferred_element_type=jnp.float32)
            fa = fa * jnp.float32(1.0 / 196.0)
            xpa_ref[b] = _quant(fa, jnp.min(fa), jnp.max(fa))
        fms = fms_ref[...]
        mn = jnp.min(fms, axis=1, keepdims=True)
        mx = jnp.max(fms, axis=1, keepdims=True)
        xms_ref[...] = _quant(fms, mn, mx)
        pa = xpa_ref[...]
        bins4 = lax.broadcasted_iota(jnp.int32, (1, 1, 1, _N), 3)
        eq4 = (pa[:, :, :, None] == bins4).astype(jnp.int32)
        cp = jnp.sum(jnp.sum(eq4, axis=2), axis=1)
        sbins = lax.broadcasted_iota(jnp.int32, (1, 1, _UW), 2)
        w = jnp.sum((cp[:, :, None] == sbins).astype(jnp.int32),
                    axis=1).astype(jnp.float32)
        s1 = lax.broadcasted_iota(jnp.int32, (_UW, _UW), 0).astype(jnp.float32)
        s2 = lax.broadcasted_iota(jnp.int32, (_UW, _UW), 1).astype(jnp.float32)
        M = _g_tilde(jnp.float32(_N) - s1 - s2)
        mw_ref[...] = lax.dot_general(w, M, (((1,), (0,)), ((), ())),
                                      preferred_element_type=jnp.float32)


def _k1(f_p, f_ms_r):
    return pl.pallas_call(
        _k1_body,
        grid=(_NST,),
        in_specs=[pl.BlockSpec((_B, _C, _HCH, _W), lambda k: (0, 0, k, 0)),
                  pl.BlockSpec(memory_space=pltpu.VMEM)],
        out_specs=[pl.BlockSpec(memory_space=pltpu.VMEM),
                   pl.BlockSpec(memory_space=pltpu.VMEM),
                   pl.BlockSpec(memory_space=pltpu.VMEM)],
        out_shape=[jax.ShapeDtypeStruct((_B, 16, 16), jnp.int32),
                   jax.ShapeDtypeStruct((_B * _C, _N), jnp.int32),
                   jax.ShapeDtypeStruct((_B, _UW), jnp.float32)],
        scratch_shapes=[pltpu.VMEM((_B, _H, _W), jnp.float32)],
    )(f_p, f_ms_r)


_NWK = 32
_RPW = (_B * _C) // _NWK
_JSZ = 65536


_RN = _RPW * _N


def _sc_body(xms_hbm, xpa_hbm, mw_hbm, a_hbm, c_hbm, main_hbm,
             xms_a, xpa_v, idxA, idxB, idxC, idxD, a_all, c_all, cmv, gv,
             mwv, main_a, onesv, negv, zerosv, semS, semG, semU, semM,
             joint_s, cm_s, mw_s):
    cid = lax.axis_index("c")
    sid = lax.axis_index("s")
    wid = sid * 2 + cid

    one16 = jnp.full((16,), 1, jnp.int32)
    zero16 = jnp.full((16,), 0, jnp.int32)
    for k in range(8):
        onesv[pl.ds(k * 16, 16)] = one16
        negv[pl.ds(k * 16, 16)] = -one16

    def zbody(i, carry):
        zerosv[pl.ds(i * 16, 16)] = zero16
        return carry
    lax.fori_loop(0, 256, zbody, 0)

    base_j = sid * _JSZ
    base_cm0 = sid * _RN

    def jz(i, carry):
        pltpu.sync_copy(zerosv, joint_s.at[pl.ds(base_j + i * 4096, 4096)])
        return carry
    lax.fori_loop(0, _JSZ // 4096, jz, 0)
    pltpu.sync_copy(zerosv.at[pl.ds(0, 3072)], cm_s.at[pl.ds(base_cm0, _RN)])

    pltpu.sync_copy(xms_hbm.at[pl.ds(wid * _RN, _RN)], xms_a)
    pltpu.sync_copy(xpa_hbm.at[pl.ds((wid // 8) * _N, _N)], xpa_v)
    mw_base = sid * _UW
    pltpu.sync_copy(mw_hbm.at[pl.ds((wid // 8) * _UW, _UW)], mwv)
    pltpu.sync_copy(mwv, mw_s.at[pl.ds(mw_base, _UW)])

    def row_body(r, carry):
        ro = r * _N
        base_cm = base_cm0 + ro
        for k in range(8):
            ms_lo = xms_a[pl.ds(ro + k * 16, 16)]
            ms_hi = xms_a[pl.ds(ro + 128 + k * 16, 16)]
            idxA[pl.ds(k * 16, 16)] = ms_lo + base_cm
            idxB[pl.ds(k * 16, 16)] = ms_hi + base_cm
            idxC[pl.ds(k * 16, 16)] = (ms_lo * 256
                                       + xpa_v[pl.ds(k * 16, 16)] + base_j)
            idxD[pl.ds(k * 16, 16)] = (ms_hi * 256
                                       + xpa_v[pl.ds(128 + k * 16, 16)] + base_j)
        s1 = pltpu.async_copy(onesv, cm_s.at[idxA], semS, add=True)
        s2 = pltpu.async_copy(onesv, cm_s.at[idxB], semS, add=True)
        s3 = pltpu.async_copy(onesv, joint_s.at[idxC], semS, add=True)
        s4 = pltpu.async_copy(onesv, joint_s.at[idxD], semS, add=True)
        s1.wait(); s2.wait(); s3.wait(); s4.wait()
        g1 = pltpu.async_copy(cm_s.at[idxA], a_all.at[pl.ds(ro, 128)], semG)
        g2 = pltpu.async_copy(cm_s.at[idxB], a_all.at[pl.ds(ro + 128, 128)],
                              semG)
        g3 = pltpu.async_copy(joint_s.at[idxC], c_all.at[pl.ds(ro, 128)],
                              semG)
        g4 = pltpu.async_copy(joint_s.at[idxD], c_all.at[pl.ds(ro + 128, 128)],
                              semG)
        g5 = pltpu.async_copy(cm_s.at[pl.ds(base_cm, _N)], cmv, semG)
        g1.wait(); g2.wait(); g3.wait(); g4.wait(); g5.wait()
        u1 = pltpu.async_copy(negv, joint_s.at[idxC], semU, add=True)
        u2 = pltpu.async_copy(negv, joint_s.at[idxD], semU, add=True)
        for k in range(8):
            idxA[pl.ds(k * 16, 16)] = cmv[pl.ds(k * 16, 16)] + mw_base
            idxB[pl.ds(k * 16, 16)] = cmv[pl.ds(128 + k * 16, 16)] + mw_base
        m1 = pltpu.async_copy(mw_s.at[idxA], gv.at[pl.ds(0, 128)], semM)
        m2 = pltpu.async_copy(mw_s.at[idxB], gv.at[pl.ds(128, 128)], semM)
        m1.wait(); m2.wait()
        accm = gv[pl.ds(0, 16)]
        for k in range(1, 16):
            accm = accm + gv[pl.ds(k * 16, 16)]
        main_a[pl.ds(r * 16, 16)] = accm
        u1.wait(); u2.wait()
        return carry
    lax.fori_loop(0, _RPW, row_body, 0)

    pltpu.sync_copy(a_all, a_hbm.at[pl.ds(wid * _RN, _RN)])
    pltpu.sync_copy(c_all, c_hbm.at[pl.ds(wid * _RN, _RN)])
    pltpu.sync_copy(main_a, main_hbm.at[pl.ds(wid * _RPW * 16, _RPW * 16)])


def _sc_counts(x_ms, x_pa, mw):
    mesh = plsc.VectorSubcoreMesh(core_axis_name="c", subcore_axis_name="s")
    f = pl.kernel(
        _sc_body,
        out_type=[jax.ShapeDtypeStruct((_B * _C * _N,), jnp.int32),
                  jax.ShapeDtypeStruct((_B * _C * _N,), jnp.int32),
                  jax.ShapeDtypeStruct((_B * _C * 16,), jnp.float32)],
        mesh=mesh,
        scratch_types=[
            pltpu.VMEM((_RN,), jnp.int32),
            pltpu.VMEM((_N,), jnp.int32),
            pltpu.VMEM((128,), jnp.int32),
            pltpu.VMEM((128,), jnp.int32),
            pltpu.VMEM((128,), jnp.int32),
            pltpu.VMEM((128,), jnp.int32),
            pltpu.VMEM((_RN,), jnp.int32),
            pltpu.VMEM((_RN,), jnp.int32),
            pltpu.VMEM((_N,), jnp.int32),
            pltpu.VMEM((_N,), jnp.float32),
            pltpu.VMEM((_UW,), jnp.float32),
            pltpu.VMEM((_RPW * 16,), jnp.float32),
            pltpu.VMEM((128,), jnp.int32),
            pltpu.VMEM((128,), jnp.int32),
            pltpu.VMEM((4096,), jnp.int32),
            pltpu.SemaphoreType.DMA,
            pltpu.SemaphoreType.DMA,
            pltpu.SemaphoreType.DMA,
            pltpu.SemaphoreType.DMA,
            pltpu.VMEM_SHARED((16 * _JSZ,), jnp.int32),
            pltpu.VMEM_SHARED((16 * _RN,), jnp.int32),
            pltpu.VMEM_SHARED((16 * _UW,), jnp.float32),
        ],
    )
    return f(x_ms, x_pa, mw)


def _final_body(a_ref, c_ref, mp_ref, xpa_ref, fms_ref, o_ref):
    af = a_ref[...].astype(jnp.float32)
    h_ms = -(1.0 / _N) * jnp.sum(jnp.log(af * (1.0 / _N) + 1e-8), axis=2)

    pa = xpa_ref[...]
    bq = jnp.sum((pa[:, :, None] == pa[:, None, :]).astype(jnp.int32), axis=2)

    main = jnp.sum(mp_ref[...], axis=2)

    cf = c_ref[...].astype(jnp.float32)
    sf = af + bq[:, None, :].astype(jnp.float32)
    histic = jnp.float32(_N) - sf + 2.0 * cf
    corr = jnp.sum((_g(histic) - _g_tilde(jnp.float32(_N) - sf)) / cf, axis=2)
    h_ms_p = -(main + corr)

    mi = h_ms - h_ms_p
    mx = jnp.max(mi, axis=1, keepdims=True)
    e = jnp.exp(mi - mx)
    soft = e / jnp.sum(e, axis=1, keepdims=True)
    o_ref[...] = fms_ref[...] * (1.0 + soft[:, :, None])


def _final(a3, c3, mp3, x_pa, f_ms3):
    return pl.pallas_call(
        _final_body,
        in_specs=[pl.BlockSpec(memory_space=pltpu.VMEM)] * 5,
        out_specs=pl.BlockSpec(memory_space=pltpu.VMEM),
        out_shape=jax.ShapeDtypeStruct((_B, _C, _N), jnp.float32),
    )(a3, c3, mp3, x_pa, f_ms3)


def kernel(f_p, f_ms):
    B, C, H, W = f_p.shape
    x_pa, x_ms, mw = _k1(f_p, f_ms.reshape(B * C, _N))
    a, c, mainp = _sc_counts(x_ms.reshape(B * C * _N),
                             x_pa.reshape(B * _N), mw.reshape(B * _UW))
    rel = _final(a.reshape(B, C, _N), c.reshape(B, C, _N),
                 mainp.reshape(B, C, 16), x_pa.reshape(B, _N),
                 f_ms.reshape(B, C, _N))
    return rel.reshape(B, C, 16, 16)
